# Initial kernel scaffold; baseline (speedup 1.0000x reference)
import jax
import jax.numpy as jnp
from jax.experimental import pallas as pl


def kernel(x, lut):
    raise NotImplementedError("write your pallas kernel here")


# speedup vs baseline: 3.8603x; 3.8603x over previous
import functools
import math

import jax
import jax.numpy as jnp
from jax import lax
from jax.experimental import pallas as pl
from jax.experimental.pallas import tpu as pltpu
from jax.experimental.pallas import tpu_sc as plsc

D_MODEL = 64
SCALE = math.sqrt(D_MODEL)

NC, NS = 2, 16
NW = NC * NS

B_TOKENS = 4096 * 200
BPW = B_TOKENS // NW
SUB = 128
K = 4
CHUNK = SUB * K
NCHUNK = BPW // CHUNK


def _scale_body(lut_ref, out_ref):
    out_ref[...] = lut_ref[...] * SCALE


def _scaled_table(lut):
    lut2 = lut.reshape(50000, 128)
    out = pl.pallas_call(
        _scale_body,
        out_shape=jax.ShapeDtypeStruct((50000, 128), jnp.float32),
        grid=(10,),
        in_specs=[pl.BlockSpec((5000, 128), lambda i: (i, 0))],
        out_specs=pl.BlockSpec((5000, 128), lambda i: (i, 0)),
    )(lut2)
    return out.reshape(100000, D_MODEL)


@functools.partial(
    pl.kernel,
    out_type=jax.ShapeDtypeStruct((B_TOKENS, D_MODEL), jnp.float32),
    mesh=plsc.VectorSubcoreMesh(core_axis_name="c", subcore_axis_name="s"),
    scratch_types=[
        pltpu.VMEM((K, SUB), jnp.int32),
        pltpu.VMEM((CHUNK, D_MODEL), jnp.float32),
        pltpu.SemaphoreType.DMA,
    ],
    compiler_params=pltpu.CompilerParams(use_tc_tiling_on_sc=False),
)
def _sc_gather(idx_hbm, tab_hbm, out_hbm, idx_v, rows_v, sem):
    wid = lax.axis_index("s") * NC + lax.axis_index("c")
    row_base = wid * (BPW // SUB)

    @pl.loop(0, NCHUNK)
    def _chunk(i):
        row_off = row_base + i * K
        pltpu.sync_copy(idx_hbm.at[pl.ds(row_off, K)], idx_v)
        copies = []
        for j in range(K):
            copies.append(
                pltpu.async_copy(
                    tab_hbm.at[idx_v.at[j]],
                    rows_v.at[pl.ds(j * SUB, SUB)],
                    sem,
                )
            )
        for c in copies:
            c.wait()
        off = (row_off) * SUB
        pltpu.sync_copy(rows_v, out_hbm.at[pl.ds(off, CHUNK)])


def kernel(x, lut):
    idx = x.reshape(B_TOKENS // SUB, SUB).astype(jnp.int32)
    tab = _scaled_table(lut)
    out = _sc_gather(idx, tab)
    return out.reshape(4096, 200, D_MODEL)


# speedup vs baseline: 4.1314x; 1.0702x over previous
import functools
import math

import jax
import jax.numpy as jnp
from jax import lax
from jax.experimental import pallas as pl
from jax.experimental.pallas import tpu as pltpu
from jax.experimental.pallas import tpu_sc as plsc

D_MODEL = 64
SCALE = math.sqrt(D_MODEL)

NC, NS = 2, 16
NW = NC * NS

B_TOKENS = 4096 * 200
BPW = B_TOKENS // NW
SUB = 128
K = 4
CHUNK = SUB * K
NCHUNK = BPW // CHUNK


def _scale_body(lut_ref, out_ref):
    out_ref[...] = lut_ref[...] * SCALE


def _scaled_table(lut):
    lut2 = lut.reshape(50000, 128)
    out = pl.pallas_call(
        _scale_body,
        out_shape=jax.ShapeDtypeStruct((50000, 128), jnp.float32),
        grid=(10,),
        in_specs=[pl.BlockSpec((5000, 128), lambda i: (i, 0))],
        out_specs=pl.BlockSpec((5000, 128), lambda i: (i, 0)),
    )(lut2)
    return out.reshape(100000, D_MODEL)


NROWS_W = BPW // SUB


@functools.partial(
    pl.kernel,
    out_type=jax.ShapeDtypeStruct((B_TOKENS, D_MODEL), jnp.float32),
    mesh=plsc.VectorSubcoreMesh(core_axis_name="c", subcore_axis_name="s"),
    scratch_types=[
        pltpu.VMEM((NROWS_W, SUB), jnp.int32),
        pltpu.VMEM((2, CHUNK, D_MODEL), jnp.float32),
        pltpu.SemaphoreType.DMA,
        pltpu.SemaphoreType.DMA,
        pltpu.SemaphoreType.DMA,
    ],
    compiler_params=pltpu.CompilerParams(use_tc_tiling_on_sc=False),
)
def _sc_gather(idx_hbm, tab_hbm, out_hbm, idx_all, rows_v, sem_g, sem_o0, sem_o1):
    wid = lax.axis_index("s") * NC + lax.axis_index("c")
    row_base = wid * NROWS_W
    sem_o = (sem_o0, sem_o1)

    pltpu.sync_copy(idx_hbm.at[pl.ds(row_base, NROWS_W)], idx_all)

    @pl.loop(0, NCHUNK, step=2)
    def _chunk(i):
        for b in range(2):
            ii = i + b
            off = (row_base + ii * K) * SUB

            @pl.when(ii >= 2)
            def _drain():
                pltpu.make_async_copy(
                    rows_v.at[b], out_hbm.at[pl.ds(off, CHUNK)], sem_o[b]
                ).wait()

            gathers = [
                pltpu.async_copy(
                    tab_hbm.at[idx_all.at[ii * K + j]],
                    rows_v.at[b].at[pl.ds(j * SUB, SUB)],
                    sem_g,
                )
                for j in range(K)
            ]
            for g in gathers:
                g.wait()
            pltpu.async_copy(rows_v.at[b], out_hbm.at[pl.ds(off, CHUNK)], sem_o[b])

    for b in range(2):
        off = (row_base + (NCHUNK - 2 + b) * K) * SUB
        pltpu.make_async_copy(
            rows_v.at[b], out_hbm.at[pl.ds(off, CHUNK)], sem_o[b]
        ).wait()


def kernel(x, lut):
    idx = x.reshape(B_TOKENS // SUB, SUB).astype(jnp.int32)
    tab = _scaled_table(lut)
    out = _sc_gather(idx, tab)
    return out.reshape(4096, 200, D_MODEL)


# speedup vs baseline: 4.1496x; 1.0044x over previous
import functools
import math

import jax
import jax.numpy as jnp
from jax import lax
from jax.experimental import pallas as pl
from jax.experimental.pallas import tpu as pltpu
from jax.experimental.pallas import tpu_sc as plsc

D_MODEL = 64
SCALE = math.sqrt(D_MODEL)

NC, NS = 2, 16
NW = NC * NS

B_TOKENS = 4096 * 200
BPW = B_TOKENS // NW
SUB = 128
K = 4
CHUNK = SUB * K
NCHUNK = BPW // CHUNK


def _scale_body(lut_ref, out_ref):
    out_ref[...] = lut_ref[...] * SCALE


def _scaled_table(lut):
    lut2 = lut.reshape(50000, 128)
    out = pl.pallas_call(
        _scale_body,
        out_shape=jax.ShapeDtypeStruct((50000, 128), jnp.float32),
        grid=(10,),
        in_specs=[pl.BlockSpec((5000, 128), lambda i: (i, 0))],
        out_specs=pl.BlockSpec((5000, 128), lambda i: (i, 0)),
    )(lut2)
    return out.reshape(100000, D_MODEL)


NROWS_W = BPW // SUB


@functools.partial(
    pl.kernel,
    out_type=jax.ShapeDtypeStruct((B_TOKENS, D_MODEL), jnp.float32),
    mesh=plsc.VectorSubcoreMesh(core_axis_name="c", subcore_axis_name="s"),
    scratch_types=[
        pltpu.VMEM((NROWS_W, SUB), jnp.int32),
        pltpu.VMEM((2, CHUNK, D_MODEL), jnp.float32),
        pltpu.SemaphoreType.DMA,
        pltpu.SemaphoreType.DMA,
        pltpu.SemaphoreType.DMA,
        pltpu.SemaphoreType.DMA,
    ],
    compiler_params=pltpu.CompilerParams(use_tc_tiling_on_sc=False),
)
def _sc_gather(idx_hbm, tab_hbm, out_hbm, idx_all, rows_v,
               sem_g0, sem_g1, sem_o0, sem_o1):
    wid = lax.axis_index("s") * NC + lax.axis_index("c")
    row_base = wid * NROWS_W
    sem_g = (sem_g0, sem_g1)
    sem_o = (sem_o0, sem_o1)

    pltpu.sync_copy(idx_hbm.at[pl.ds(row_base, NROWS_W)], idx_all)

    def fire_gathers(ii, b):
        for j in range(K):
            pltpu.async_copy(
                tab_hbm.at[idx_all.at[ii * K + j]],
                rows_v.at[b].at[pl.ds(j * SUB, SUB)],
                sem_g[b],
            )

    def out_slice(ii):
        return out_hbm.at[pl.ds((row_base + ii * K) * SUB, CHUNK)]

    fire_gathers(0, 0)

    @pl.loop(0, NCHUNK, step=2)
    def _chunk(i):
        for b in range(2):
            ii = i + b
            b2 = 1 - b

            @pl.when(jnp.logical_and(ii >= 1, ii + 1 < NCHUNK))
            def _drain_out():
                pltpu.make_async_copy(rows_v.at[b2], out_slice(ii), sem_o[b2]).wait()

            @pl.when(ii + 1 < NCHUNK)
            def _next_gathers():
                fire_gathers(ii + 1, b2)

            pltpu.make_async_copy(rows_v.at[b], out_slice(ii), sem_g[b]).wait()
            pltpu.async_copy(rows_v.at[b], out_slice(ii), sem_o[b])

    for b in range(2):
        pltpu.make_async_copy(rows_v.at[b], out_slice(b), sem_o[b]).wait()


def kernel(x, lut):
    idx = x.reshape(B_TOKENS // SUB, SUB).astype(jnp.int32)
    tab = _scaled_table(lut)
    out = _sc_gather(idx, tab)
    return out.reshape(4096, 200, D_MODEL)
